# baseline (device time: 16668 ns/iter reference)
import jax
import jax.numpy as jnp
from jax import lax
from jax.experimental import pallas as pl
from jax.experimental.pallas import tpu as pltpu

N_DEV = 8
B = 2
SQ = 128
SKV_SH = 128
HQ = 4
DH = 64
HD = HQ * DH
WINDOW = 128
SCALE = 0.125
NEG = -1e9
N_LIVE = 2

SEND = {0: (0, [4, 1, 3, 2]), 1: (1, [5, 0, 2, 3])}
FWD = {4: (0, [5, 7, 6]), 5: (1, [4, 6, 7])}
ORIGIN = {0: 0, 1: 1}
RELAY = {0: 4, 1: 5}

RECV_FROM = {
    0: [1], 1: [0], 2: [0, 1], 3: [0, 1],
    4: [0, 5], 5: [1, 4], 6: [4, 5], 7: [4, 5],
}
N_CREDITS = {pos: len(t) for pos, (_, t) in (SEND | FWD).items()}


def kernel(x, Wq, K_ext, V_ext, Wo):
    k2 = K_ext.reshape(B, SKV_SH, HD).astype(jnp.bfloat16)
    v2 = V_ext.reshape(B, SKV_SH, HD).astype(jnp.bfloat16)
    d_model = x.shape[-1]

    def body(x_ref, wq_ref, k_ref, v_ref, wo_ref, out_ref,
             kv_k, kv_v, send_sems, recv_sems):
        my = lax.axis_index("i")

        def rdma(src, dst_buf, chunk, kind, tgt):
            return pltpu.make_async_remote_copy(
                src_ref=src,
                dst_ref=dst_buf.at[chunk],
                send_sem=send_sems.at[tgt, kind],
                recv_sem=recv_sems.at[chunk, kind],
                device_id=(tgt,),
                device_id_type=pl.DeviceIdType.MESH,
            )

        def recv_descr(chunk, kind):
            buf = kv_k if kind == 0 else kv_v
            return rdma(k_ref if kind == 0 else v_ref, buf, chunk, kind, 0)

        barrier_sem = pltpu.get_barrier_semaphore()
        for pos, senders in RECV_FROM.items():

            @pl.when(my == pos)
            def _(senders=senders):
                for s in senders:
                    pl.semaphore_signal(
                        barrier_sem, inc=1,
                        device_id=(s,), device_id_type=pl.DeviceIdType.MESH,
                    )
        for pos, n in N_CREDITS.items():

            @pl.when(my == pos)
            def _(n=n):
                pl.semaphore_wait(barrier_sem, n)

        for pos, (chunk, targets) in SEND.items():

            @pl.when(my == pos)
            def _(chunk=chunk, targets=targets):
                rdmas = []
                for kind, src in ((0, k_ref), (1, v_ref)):
                    for tgt in targets:
                        r = rdma(src, kv_k if kind == 0 else kv_v,
                                 chunk, kind, tgt)
                        r.start()
                        rdmas.append(r)
                kv_k[chunk] = k_ref[...]
                kv_v[chunk] = v_ref[...]
                for r in rdmas:
                    r.wait_send()

        for pos, (chunk, targets) in FWD.items():

            @pl.when(my == pos)
            def _(chunk=chunk, targets=targets):
                rdmas = []
                for kind in (0, 1):
                    recv_descr(chunk, kind).wait_recv()
                    buf = kv_k if kind == 0 else kv_v
                    for tgt in targets:
                        r = rdma(buf.at[chunk], buf, chunk, kind, tgt)
                        r.start()
                        rdmas.append(r)
                for r in rdmas:
                    r.wait_send()

        qs = [
            jnp.dot(
                x_ref[b].astype(jnp.bfloat16),
                wq_ref[...].astype(jnp.bfloat16),
                preferred_element_type=jnp.float32,
            )
            for b in range(B)
        ]

        for chunk in range(N_LIVE):

            @pl.when((my != ORIGIN[chunk]) & (my != RELAY[chunk]))
            def _(chunk=chunk):
                for kind in (0, 1):
                    recv_descr(chunk, kind).wait_recv()

        tril = (
            lax.broadcasted_iota(jnp.int32, (SQ, SKV_SH), 1)
            <= lax.broadcasted_iota(jnp.int32, (SQ, SKV_SH), 0)
        )

        for b in range(B):
            k0, k1 = kv_k[0, b], kv_k[1, b]
            v0, v1 = kv_v[0, b], kv_v[1, b]
            ctx_parts = []
            for h in range(HQ):
                sl = slice(h * DH, (h + 1) * DH)
                qh = qs[b][:, sl].astype(jnp.bfloat16)
                dims = (((1,), (1,)), ((), ()))
                s0 = lax.dot_general(
                    qh, k0[:, sl], dims, preferred_element_type=jnp.float32
                )
                s1 = lax.dot_general(
                    qh, k1[:, sl], dims, preferred_element_type=jnp.float32
                )
                e0 = jnp.exp(s0 * SCALE)
                e1 = jnp.where(tril, jnp.exp(s1 * SCALE), 0.0)
                denom = jnp.sum(e0, axis=1, keepdims=True) + jnp.sum(
                    e1, axis=1, keepdims=True
                )
                num = jnp.dot(
                    e0.astype(jnp.bfloat16), v0[:, sl],
                    preferred_element_type=jnp.float32,
                ) + jnp.dot(
                    e1.astype(jnp.bfloat16), v1[:, sl],
                    preferred_element_type=jnp.float32,
                )
                ctx_parts.append(num / denom)
            ctx = jnp.concatenate(ctx_parts, axis=1)
            out_ref[b] = jnp.dot(
                ctx.astype(jnp.bfloat16),
                wo_ref[...].astype(jnp.bfloat16),
                preferred_element_type=jnp.float32,
            )

    return pl.pallas_call(
        body,
        out_shape=jax.ShapeDtypeStruct((B, SQ, d_model), jnp.float32),
        in_specs=[pl.BlockSpec(memory_space=pltpu.VMEM)] * 5,
        out_specs=pl.BlockSpec(memory_space=pltpu.VMEM),
        scratch_shapes=[
            pltpu.VMEM((N_LIVE, B, SKV_SH, HD), jnp.bfloat16),
            pltpu.VMEM((N_LIVE, B, SKV_SH, HD), jnp.bfloat16),
            pltpu.SemaphoreType.DMA((N_DEV, 2)),
            pltpu.SemaphoreType.DMA((N_LIVE, 2)),
        ],
        compiler_params=pltpu.CompilerParams(collective_id=0),
    )(x, Wq, k2, v2, Wo)


# device time: 15322 ns/iter; 1.0878x vs baseline; 1.0878x over previous
import jax
import jax.numpy as jnp
from jax import lax
from jax.experimental import pallas as pl
from jax.experimental.pallas import tpu as pltpu

N_DEV = 8
B = 2
SQ = 128
SKV_SH = 128
HQ = 4
DH = 64
HD = HQ * DH
WINDOW = 128
SCALE = 0.125
NEG = -1e9
N_LIVE = 2

SEND = {0: (0, [4, 1, 3, 2]), 1: (1, [5, 0, 2, 3])}
FWD = {4: (0, [5, 7, 6]), 5: (1, [4, 6, 7])}
ORIGIN = {0: 0, 1: 1}
RELAY = {0: 4, 1: 5}

RECV_FROM = {
    0: [1], 1: [0], 2: [0, 1], 3: [0, 1],
    4: [0, 5], 5: [1, 4], 6: [4, 5], 7: [4, 5],
}
N_CREDITS = {pos: len(t) for pos, (_, t) in (SEND | FWD).items()}


def kernel(x, Wq, K_ext, V_ext, Wo):
    k2 = K_ext.reshape(B, SKV_SH, HD).astype(jnp.bfloat16)
    v2 = V_ext.reshape(B, SKV_SH, HD).astype(jnp.bfloat16)
    d_model = x.shape[-1]

    def body(x_ref, wq_ref, k_ref, v_ref, wo_ref, out_ref,
             kv_k, kv_v, send_sems, recv_sems):
        my = lax.axis_index("i")

        def rdma(src, dst_buf, chunk, kind, tgt):
            return pltpu.make_async_remote_copy(
                src_ref=src,
                dst_ref=dst_buf.at[chunk],
                send_sem=send_sems.at[tgt, kind],
                recv_sem=recv_sems.at[chunk, kind],
                device_id=(tgt,),
                device_id_type=pl.DeviceIdType.MESH,
            )

        def recv_descr(chunk, kind):
            buf = kv_k if kind == 0 else kv_v
            return rdma(k_ref if kind == 0 else v_ref, buf, chunk, kind, 0)

        barrier_sem = pltpu.get_barrier_semaphore()
        for pos, senders in RECV_FROM.items():

            @pl.when(my == pos)
            def _(senders=senders):
                for s in senders:
                    pl.semaphore_signal(
                        barrier_sem, inc=1,
                        device_id=(s,), device_id_type=pl.DeviceIdType.MESH,
                    )
        for pos, n in N_CREDITS.items():

            @pl.when(my == pos)
            def _(n=n):
                pl.semaphore_wait(barrier_sem, n)

        for pos, (chunk, targets) in SEND.items():

            @pl.when(my == pos)
            def _(chunk=chunk, targets=targets):
                rdmas = []
                for kind, src in ((0, k_ref), (1, v_ref)):
                    for tgt in targets:
                        r = rdma(src, kv_k if kind == 0 else kv_v,
                                 chunk, kind, tgt)
                        r.start()
                        rdmas.append(r)
                kv_k[chunk] = k_ref[...]
                kv_v[chunk] = v_ref[...]

        for pos, (chunk, targets) in FWD.items():

            @pl.when(my == pos)
            def _(chunk=chunk, targets=targets):
                for kind in (0, 1):
                    recv_descr(chunk, kind).wait_recv()
                    buf = kv_k if kind == 0 else kv_v
                    for tgt in targets:
                        rdma(buf.at[chunk], buf, chunk, kind, tgt).start()

        qs = [
            jnp.dot(
                x_ref[b].astype(jnp.bfloat16),
                wq_ref[...].astype(jnp.bfloat16),
                preferred_element_type=jnp.float32,
            )
            for b in range(B)
        ]

        for chunk in range(N_LIVE):

            @pl.when((my != ORIGIN[chunk]) & (my != RELAY[chunk]))
            def _(chunk=chunk):
                for kind in (0, 1):
                    recv_descr(chunk, kind).wait_recv()

        n_keys = N_LIVE * SKV_SH
        qi = lax.broadcasted_iota(jnp.int32, (SQ, n_keys), 0)
        kj = lax.broadcasted_iota(jnp.int32, (SQ, n_keys), 1)
        mask = jnp.abs(qi - kj) <= WINDOW

        for b in range(B):
            kb = jnp.concatenate([kv_k[0, b], kv_k[1, b]], axis=0)
            vb = jnp.concatenate([kv_v[0, b], kv_v[1, b]], axis=0)
            ctx_parts = []
            for h in range(HQ):
                sl = slice(h * DH, (h + 1) * DH)
                qh = qs[b][:, sl].astype(jnp.bfloat16)
                kh = kb[:, sl]
                vh = vb[:, sl]
                s = lax.dot_general(
                    qh, kh, (((1,), (1,)), ((), ())),
                    preferred_element_type=jnp.float32,
                ) * SCALE
                s = jnp.where(mask, s, NEG)
                m = jnp.max(s, axis=1, keepdims=True)
                w = jnp.exp(s - m)
                w = (w / jnp.sum(w, axis=1, keepdims=True)).astype(
                    jnp.bfloat16
                )
                ctx_parts.append(
                    jnp.dot(w, vh, preferred_element_type=jnp.float32)
                )
            ctx = jnp.concatenate(ctx_parts, axis=1)
            out_ref[b] = jnp.dot(
                ctx, wo_ref[...], preferred_element_type=jnp.float32
            )

        for pos, (chunk, targets) in SEND.items():

            @pl.when(my == pos)
            def _(chunk=chunk, targets=targets):
                for kind, src in ((0, k_ref), (1, v_ref)):
                    for tgt in targets:
                        rdma(src, kv_k if kind == 0 else kv_v,
                             chunk, kind, tgt).wait_send()

        for pos, (chunk, targets) in FWD.items():

            @pl.when(my == pos)
            def _(chunk=chunk, targets=targets):
                for kind in (0, 1):
                    buf = kv_k if kind == 0 else kv_v
                    for tgt in targets:
                        rdma(buf.at[chunk], buf, chunk, kind, tgt).wait_send()

    return pl.pallas_call(
        body,
        out_shape=jax.ShapeDtypeStruct((B, SQ, d_model), jnp.float32),
        in_specs=[pl.BlockSpec(memory_space=pltpu.VMEM)] * 5,
        out_specs=pl.BlockSpec(memory_space=pltpu.VMEM),
        scratch_shapes=[
            pltpu.VMEM((N_LIVE, B, SKV_SH, HD), jnp.bfloat16),
            pltpu.VMEM((N_LIVE, B, SKV_SH, HD), jnp.bfloat16),
            pltpu.SemaphoreType.DMA((N_DEV, 2)),
            pltpu.SemaphoreType.DMA((N_LIVE, 2)),
        ],
        compiler_params=pltpu.CompilerParams(collective_id=0),
    )(x, Wq, k2, v2, Wo)
